# baseline (speedup 1.0000x reference)
import functools

import numpy as np
import jax
import jax.numpy as jnp
from jax import lax
from jax.experimental import pallas as pl
from jax.experimental.pallas import tpu as pltpu
from jax.experimental.pallas import tpu_sc as plsc

_RES = 6
_GMIN = -2.0
_GMAX = 2.0
_STEP = (_GMAX - _GMIN) / (_RES - 1)

_G_HOST = np.zeros((1024, 128), np.float32)
for _j in range(1024):
    _G_HOST[_j, _j // 8] = float(_RES ** (7 - (_j % 8)))

_IDX_BLOCK = 512


def _quant_body(x_ref, g_ref, line_ref, rem_ref):
    xc = jnp.clip(x_ref[...], _GMIN, _GMAX)
    t = jnp.round((xc - _GMIN) / _STEP)
    f = lax.dot_general(
        t, g_ref[...], (((1,), (0,)), ((), ())),
        precision=lax.Precision.HIGHEST,
        preferred_element_type=jnp.float32)
    idx = f.astype(jnp.int32)
    line_ref[...] = idx >> 4
    rem_ref[...] = idx & 15


def _quantize(x2d, g):
    m, n = x2d.shape
    grid = m // _IDX_BLOCK
    out_sds = jax.ShapeDtypeStruct((m, n // 8), jnp.int32)
    return pl.pallas_call(
        _quant_body,
        grid=(grid,),
        in_specs=[
            pl.BlockSpec((_IDX_BLOCK, n), lambda i: (i, 0)),
            pl.BlockSpec((n, n // 8), lambda i: (0, 0)),
        ],
        out_specs=[
            pl.BlockSpec((_IDX_BLOCK, n // 8), lambda i: (i, 0)),
            pl.BlockSpec((_IDX_BLOCK, n // 8), lambda i: (i, 0)),
        ],
        out_shape=[out_sds, out_sds],
        compiler_params=pltpu.CompilerParams(
            dimension_semantics=("parallel",)),
    )(x2d, g)


_NC = 2
_NS = 16
_NW = _NC * _NS
_CH = 8


def _gather_sc(table_lines, idx_line, idx_rem):
    nrows = idx_line.shape[0]
    rpw = nrows // _NW
    out_lines = nrows * 8
    olpc = _CH * 8
    mesh = plsc.VectorSubcoreMesh(core_axis_name="c", subcore_axis_name="s")

    @functools.partial(
        pl.kernel,
        mesh=mesh,
        out_type=jax.ShapeDtypeStruct((out_lines, 128), jnp.float32),
        scratch_types=[
            pltpu.VMEM((_CH, 128), jnp.int32),
            pltpu.VMEM((_CH, 128), jnp.int32),
            pltpu.VMEM((128, 128), jnp.float32),
            pltpu.VMEM((128, 128), jnp.float32),
            pltpu.VMEM((olpc, 128), jnp.float32),
            pltpu.SemaphoreType.DMA,
            pltpu.SemaphoreType.DMA,
        ],
        compiler_params=pltpu.CompilerParams(
            use_tc_tiling_on_sc=True, needs_layout_passes=False),
    )
    def gather_kernel(table_hbm, line_hbm, rem_hbm, out_hbm,
                      line_v, rem_v, lines_a, lines_b, obuf, sem_a, sem_b):
        wid = lax.axis_index("s") * _NC + lax.axis_index("c")
        iota = lax.iota(jnp.int32, 16)
        bufs = (lines_a, lines_b)
        sems = (sem_a, sem_b)

        def extract(j, lines_v):
            jv = jnp.full((16,), j, jnp.int32)
            for k in range(1024 // 16):
                pvec = iota + 16 * k
                r = pvec >> 3
                c = pvec & 7
                qrem = plsc.load_gather(rem_v, [jv, r])
                col = (qrem << 3) | c
                vals = plsc.load_gather(lines_v, [r, col])
                obuf[8 * j + k // 8, pl.ds((k % 8) * 16, 16)] = vals

        @pl.loop(0, rpw // _CH)
        def _chunk(ch):
            ibase = wid * rpw + ch * _CH
            pltpu.sync_copy(line_hbm.at[pl.ds(ibase, _CH)], line_v)
            pltpu.sync_copy(rem_hbm.at[pl.ds(ibase, _CH)], rem_v)
            copies = [None] * _CH
            for j in range(_CH):
                copies[j] = pltpu.async_copy(
                    table_hbm.at[line_v.at[j]], bufs[j % 2], sems[j % 2])
                if j > 0:
                    copies[j - 1].wait()
                    extract(j - 1, bufs[(j - 1) % 2])
            copies[_CH - 1].wait()
            extract(_CH - 1, bufs[(_CH - 1) % 2])
            obase = wid * (rpw * 8) + ch * olpc
            pltpu.sync_copy(obuf, out_hbm.at[pl.ds(obase, olpc)])

    return gather_kernel(table_lines, idx_line, idx_rem)


def kernel(x, table):
    b, s, d = x.shape
    x2d = x.reshape(b // 2, 2 * s * d)
    g = jnp.asarray(_G_HOST)
    idx_line, idx_rem = _quantize(x2d, g)
    table_lines = table.reshape(-1).astype(jnp.float32).reshape(104976, 128)
    y = _gather_sc(table_lines, idx_line, idx_rem)
    return y.reshape(x.shape)


# speedup vs baseline: 1.0309x; 1.0309x over previous
import functools

import numpy as np
import jax
import jax.numpy as jnp
from jax import lax
from jax.experimental import pallas as pl
from jax.experimental.pallas import tpu as pltpu
from jax.experimental.pallas import tpu_sc as plsc

_RES = 6
_GMIN = -2.0
_GMAX = 2.0
_STEP = (_GMAX - _GMIN) / (_RES - 1)

_G_HOST = np.zeros((1024, 128), np.float32)
for _j in range(1024):
    _G_HOST[_j, _j // 8] = float(_RES ** (7 - (_j % 8)))

_IDX_BLOCK = 512


def _quant_body(x_ref, g_ref, line_ref, rem_ref):
    xc = jnp.clip(x_ref[...], _GMIN, _GMAX)
    t = jnp.round((xc - _GMIN) / _STEP)
    f = lax.dot_general(
        t, g_ref[...], (((1,), (0,)), ((), ())),
        precision=lax.Precision.HIGHEST,
        preferred_element_type=jnp.float32)
    idx = f.astype(jnp.int32)
    line_ref[...] = idx >> 4
    rem_ref[...] = idx & 15


def _quantize(x2d, g):
    m, n = x2d.shape
    grid = m // _IDX_BLOCK
    out_sds = jax.ShapeDtypeStruct((m, n // 8), jnp.int32)
    return pl.pallas_call(
        _quant_body,
        grid=(grid,),
        in_specs=[
            pl.BlockSpec((_IDX_BLOCK, n), lambda i: (i, 0)),
            pl.BlockSpec((n, n // 8), lambda i: (0, 0)),
        ],
        out_specs=[
            pl.BlockSpec((_IDX_BLOCK, n // 8), lambda i: (i, 0)),
            pl.BlockSpec((_IDX_BLOCK, n // 8), lambda i: (i, 0)),
        ],
        out_shape=[out_sds, out_sds],
        compiler_params=pltpu.CompilerParams(
            dimension_semantics=("parallel",)),
    )(x2d, g)


_NC = 2
_NS = 16
_NW = _NC * _NS
_CH = 16
_NB = 4


def _gather_sc(table_lines, idx_line, idx_rem):
    nrows = idx_line.shape[0]
    rpw = nrows // _NW
    out_lines = nrows * 8
    olpc = _CH * 8
    mesh = plsc.VectorSubcoreMesh(core_axis_name="c", subcore_axis_name="s")

    @functools.partial(
        pl.kernel,
        mesh=mesh,
        out_type=jax.ShapeDtypeStruct((out_lines, 128), jnp.float32),
        scratch_types=[
            pltpu.VMEM((_CH, 128), jnp.int32),
            pltpu.VMEM((_CH, 128), jnp.int32),
        ] + [pltpu.VMEM((128, 128), jnp.float32)] * _NB + [
            pltpu.VMEM((olpc, 128), jnp.float32),
        ] + [pltpu.SemaphoreType.DMA] * _NB + [
            pltpu.SemaphoreType.DMA,
        ],
        compiler_params=pltpu.CompilerParams(
            use_tc_tiling_on_sc=True, needs_layout_passes=False),
    )
    def gather_kernel(table_hbm, line_hbm, rem_hbm, out_hbm,
                      line_v, rem_v, b0, b1, b2, b3, obuf,
                      s0, s1, s2, s3, osem):
        wid = lax.axis_index("s") * _NC + lax.axis_index("c")
        iota = lax.iota(jnp.int32, 16)
        bufs = (b0, b1, b2, b3)
        sems = (s0, s1, s2, s3)
        first_blk = out_hbm.at[pl.ds(wid * (rpw * 8), olpc)]

        def extract(j, lines_v):
            jv = jnp.full((16,), j, jnp.int32)
            for k in range(1024 // 16):
                pvec = iota + 16 * k
                r = pvec >> 3
                c = pvec & 7
                qrem = plsc.load_gather(rem_v, [jv, r])
                col = (qrem << 3) | c
                vals = plsc.load_gather(lines_v, [r, col])
                obuf[8 * j + k // 8, pl.ds((k % 8) * 16, 16)] = vals

        pltpu.async_copy(obuf, first_blk, osem)

        @pl.loop(0, rpw // _CH)
        def _chunk(ch):
            ibase = wid * rpw + ch * _CH
            pltpu.sync_copy(line_hbm.at[pl.ds(ibase, _CH)], line_v)
            pltpu.sync_copy(rem_hbm.at[pl.ds(ibase, _CH)], rem_v)
            copies = [None] * _CH
            for j in range(_NB):
                copies[j] = pltpu.async_copy(
                    table_hbm.at[line_v.at[j]], bufs[j % _NB], sems[j % _NB])
            pltpu.make_async_copy(obuf, first_blk, osem).wait()
            for j in range(_CH):
                copies[j].wait()
                extract(j, bufs[j % _NB])
                if j + _NB < _CH:
                    copies[j + _NB] = pltpu.async_copy(
                        table_hbm.at[line_v.at[j + _NB]],
                        bufs[(j + _NB) % _NB], sems[(j + _NB) % _NB])
            obase = wid * (rpw * 8) + ch * olpc
            pltpu.async_copy(obuf, out_hbm.at[pl.ds(obase, olpc)], osem)

        pltpu.make_async_copy(obuf, first_blk, osem).wait()

    return gather_kernel(table_lines, idx_line, idx_rem)


def kernel(x, table):
    b, s, d = x.shape
    x2d = x.reshape(b // 2, 2 * s * d)
    g = jnp.asarray(_G_HOST)
    idx_line, idx_rem = _quantize(x2d, g)
    table_lines = table.reshape(-1).astype(jnp.float32).reshape(104976, 128)
    y = _gather_sc(table_lines, idx_line, idx_rem)
    return y.reshape(x.shape)
